# baseline (speedup 1.0000x reference)
import functools

import jax
import jax.numpy as jnp
import numpy as np
from jax import lax
from jax.experimental import pallas as pl
from jax.experimental.pallas import tpu as pltpu
from jax.experimental.pallas import tpu_sc as plsc

_NUM_BINS = 36
_MIN_ACTIVATION = 0.1
_PENALTY_WEIGHT = 1.0
_H = 384
_W = 384
_B = 64
_NTILES = 32
_ROWS_PER_DMA = 64
_NDMA = _H // _ROWS_PER_DMA

_COT = tuple(
    float(np.float32(np.cos((j - 18) * np.pi / 18.0) / np.sin((j - 18) * np.pi / 18.0)))
    for j in range(1, 18)
)


def _take16(v, idx):
    return lax.gather(
        v,
        idx[:, None],
        lax.GatherDimensionNumbers(
            offset_dims=(), collapsed_slice_dims=(0,), start_index_map=(0,)
        ),
        slice_sizes=(1,),
        mode=lax.GatherScatterMode.PROMISE_IN_BOUNDS,
    )


def _cumsum_body(m_ref, p_ref):
    x = m_ref[0] - jnp.float32(_MIN_ACTIVATION)
    r = lax.broadcasted_iota(jnp.int32, (_W, _W), 0)
    c = lax.broadcasted_iota(jnp.int32, (_W, _W), 1)
    u = (r <= c).astype(jnp.float32)
    p_ref[...] = lax.dot_general(
        x,
        u,
        (((1,), (0,)), ((), ())),
        precision=lax.Precision.HIGHEST,
        preferred_element_type=jnp.float32,
    )


def _prefix_sums(m):
    return pl.pallas_call(
        _cumsum_body,
        grid=(_B,),
        in_specs=[pl.BlockSpec((1, _H, _W), lambda b: (b, 0, 0))],
        out_specs=pl.BlockSpec((_H, _W), lambda b: (b, 0)),
        out_shape=jax.ShapeDtypeStruct((_B * _H, _W), jnp.float32),
    )(m)


def _sc_body(p_hbm, cc_hbm, out_hbm, cc_v, pbuf, acc_hi, acc_lo, pen_v):
    wid = lax.axis_index("s") * 2 + lax.axis_index("c")
    lane = lax.iota(jnp.int32, 16)
    zeros16 = jnp.zeros((16,), jnp.float32)

    pltpu.sync_copy(cc_hbm, cc_v)

    pen_vec = zeros16
    for si in range(2):
        b = wid * 2 + si
        bl = jnp.zeros((16,), jnp.int32) + (b % 16)
        cx = _take16(cc_v[pl.ds((b // 16) * 16, 16)], bl)
        cy = _take16(cc_v[pl.ds(_B + (b // 16) * 16, 16)], bl)

        for j in range(18):
            acc_hi[pl.ds(j * 16, 16)] = zeros16
            acc_lo[pl.ds(j * 16, 16)] = zeros16

        def group_body(g, _):
            pltpu.sync_copy(
                p_hbm.at[pl.ds(b * _H + g * _ROWS_PER_DMA, _ROWS_PER_DMA), :], pbuf
            )
            for s in range(_ROWS_PER_DMA // 16):
                row = s * 16 + lane
                yv = (g * _ROWS_PER_DMA + s * 16 + lane).astype(jnp.float32)
                dy = yv - cy
                use_floor = dy > 0.0
                high = dy >= 0.0
                tot = plsc.load_gather(pbuf, [row, jnp.zeros((16,), jnp.int32) + (_W - 1)])
                g_prev = jnp.where(high, tot, 0.0)
                for j in range(1, 19):
                    if j < 18:
                        wj = cx + dy * _COT[j - 1]
                        t0 = wj.astype(jnp.int32)
                        tf = t0.astype(jnp.float32)
                        ifl = t0 - (tf > wj).astype(jnp.int32)
                        icl = t0 + (tf < wj).astype(jnp.int32) - 1
                        idx = jnp.clip(jnp.where(use_floor, ifl, icl), -1, _W - 1)
                        gj = plsc.load_gather(pbuf, [row, jnp.maximum(idx, 0)])
                        gj = jnp.where(idx < 0, 0.0, gj)
                    else:
                        gj = jnp.where(high, 0.0, tot)
                    d = g_prev - gj
                    plsc.addupdate(acc_hi.at[pl.ds((j - 1) * 16, 16)], jnp.where(high, d, 0.0))
                    plsc.addupdate(acc_lo.at[pl.ds((j - 1) * 16, 16)], jnp.where(high, 0.0, -d))
                    g_prev = gj
            return _

        lax.fori_loop(0, _NDMA, group_body, 0)

        last = jnp.zeros((16,), jnp.int32) + 15
        under_vec = zeros16
        for acc in (acc_hi, acc_lo):
            for j in range(18):
                u = _take16(plsc.cumsum(acc[pl.ds(j * 16, 16)]), last)
                under_vec += jnp.where(u <= 0.0, 1.0, 0.0)
        pen_vec = jnp.where(lane == si, under_vec / jnp.float32(_NUM_BINS), pen_vec)

    pen_v[...] = pen_vec
    pltpu.sync_copy(pen_v, out_hbm.at[wid])


@functools.partial(
    pl.kernel,
    mesh=plsc.VectorSubcoreMesh(core_axis_name="c", subcore_axis_name="s"),
    out_type=jax.ShapeDtypeStruct((_NTILES, 16), jnp.float32),
    scratch_types=[
        pltpu.VMEM((2 * _B,), jnp.float32),
        pltpu.VMEM((_ROWS_PER_DMA, _W), jnp.float32),
        pltpu.VMEM((18 * 16,), jnp.float32),
        pltpu.VMEM((18 * 16,), jnp.float32),
        pltpu.VMEM((16,), jnp.float32),
    ],
    compiler_params=pltpu.CompilerParams(needs_layout_passes=False),
)
def _sc_bins(p_hbm, cc_hbm, out_hbm, cc_v, pbuf, acc_hi, acc_lo, pen_v):
    _sc_body(p_hbm, cc_hbm, out_hbm, cc_v, pbuf, acc_hi, acc_lo, pen_v)


def kernel(mask, bbox):
    m = mask.reshape(_B, _H, _W)
    cx = bbox[:, 0] * _W
    cy = bbox[:, 1] * _H
    cc = jnp.stack([cx, cy])
    p = _prefix_sums(m)
    pens = _sc_bins(p, cc.reshape(2 * _B))
    return _PENALTY_WEIGHT * (jnp.sum(pens) / jnp.float32(_B))


# speedup vs baseline: 1.0999x; 1.0999x over previous
import functools

import jax
import jax.numpy as jnp
import numpy as np
from jax import lax
from jax.experimental import pallas as pl
from jax.experimental.pallas import tpu as pltpu
from jax.experimental.pallas import tpu_sc as plsc

_NUM_BINS = 36
_MIN_ACTIVATION = 0.1
_PENALTY_WEIGHT = 1.0
_H = 384
_W = 384
_B = 64
_NTILES = 32
_HALF = _B // 2
_ROWS_PER_DMA = 64
_NDMA = _H // _ROWS_PER_DMA

_COT = tuple(
    float(np.float32(np.cos((j - 18) * np.pi / 18.0) / np.sin((j - 18) * np.pi / 18.0)))
    for j in range(1, 18)
)


def _take16(v, idx):
    return lax.gather(
        v,
        idx[:, None],
        lax.GatherDimensionNumbers(
            offset_dims=(), collapsed_slice_dims=(0,), start_index_map=(0,)
        ),
        slice_sizes=(1,),
        mode=lax.GatherScatterMode.PROMISE_IN_BOUNDS,
    )


def _cumsum_body(m_ref, p_ref):
    x = m_ref[0] - jnp.float32(_MIN_ACTIVATION)
    r = lax.broadcasted_iota(jnp.int32, (_W, _W), 0)
    c = lax.broadcasted_iota(jnp.int32, (_W, _W), 1)
    u = (r <= c).astype(jnp.float32)
    p_ref[...] = lax.dot_general(
        x,
        u,
        (((1,), (0,)), ((), ())),
        precision=lax.Precision.HIGHEST,
        preferred_element_type=jnp.float32,
    )


def _prefix_sums(m, base):
    return pl.pallas_call(
        _cumsum_body,
        grid=(_HALF,),
        in_specs=[pl.BlockSpec((1, _H, _W), lambda b: (base + b, 0, 0))],
        out_specs=pl.BlockSpec((_H, _W), lambda b: (b, 0)),
        out_shape=jax.ShapeDtypeStruct((_HALF * _H, _W), jnp.float32),
    )(m)


def _sc_body(base, p_hbm, cc_hbm, out_hbm, cc_v, pbuf, acc_hi, acc_lo, pen_v):
    wid = lax.axis_index("s") * 2 + lax.axis_index("c")
    lane = lax.iota(jnp.int32, 16)
    zeros16 = jnp.zeros((16,), jnp.float32)

    pltpu.sync_copy(cc_hbm, cc_v)

    b = base + wid
    bl = jnp.zeros((16,), jnp.int32) + (b % 16)
    cx = _take16(cc_v[pl.ds((b // 16) * 16, 16)], bl)
    cy = _take16(cc_v[pl.ds(_B + (b // 16) * 16, 16)], bl)

    for j in range(18):
        acc_hi[pl.ds(j * 16, 16)] = zeros16
        acc_lo[pl.ds(j * 16, 16)] = zeros16

    def group_body(g, carry):
        pltpu.sync_copy(
            p_hbm.at[pl.ds(wid * _H + g * _ROWS_PER_DMA, _ROWS_PER_DMA), :], pbuf
        )
        for s in range(_ROWS_PER_DMA // 16):
            row = s * 16 + lane
            yv = (g * _ROWS_PER_DMA + s * 16 + lane).astype(jnp.float32)
            dy = yv - cy
            use_floor = dy > 0.0
            high = dy >= 0.0
            tot = plsc.load_gather(pbuf, [row, jnp.zeros((16,), jnp.int32) + (_W - 1)])
            g_prev = jnp.where(high, tot, 0.0)
            for j in range(1, 19):
                if j < 18:
                    wj = cx + dy * _COT[j - 1]
                    t0 = wj.astype(jnp.int32)
                    tf = t0.astype(jnp.float32)
                    ifl = t0 - (tf > wj).astype(jnp.int32)
                    icl = t0 + (tf < wj).astype(jnp.int32) - 1
                    idx = jnp.clip(jnp.where(use_floor, ifl, icl), -1, _W - 1)
                    gj = plsc.load_gather(pbuf, [row, jnp.maximum(idx, 0)])
                    gj = jnp.where(idx < 0, 0.0, gj)
                else:
                    gj = jnp.where(high, 0.0, tot)
                d = g_prev - gj
                plsc.addupdate(acc_hi.at[pl.ds((j - 1) * 16, 16)], jnp.where(high, d, 0.0))
                plsc.addupdate(acc_lo.at[pl.ds((j - 1) * 16, 16)], jnp.where(high, 0.0, -d))
                g_prev = gj
        return carry

    lax.fori_loop(0, _NDMA, group_body, 0)

    last = jnp.zeros((16,), jnp.int32) + 15
    under_vec = zeros16
    for acc in (acc_hi, acc_lo):
        for j in range(18):
            u = _take16(plsc.cumsum(acc[pl.ds(j * 16, 16)]), last)
            under_vec += jnp.where(u <= 0.0, 1.0, 0.0)
    pen_v[...] = jnp.where(lane == 0, under_vec / jnp.float32(_NUM_BINS), zeros16)
    pltpu.sync_copy(pen_v, out_hbm.at[wid])


def _make_sc(base):
    return pl.kernel(
        functools.partial(_sc_body, base),
        mesh=plsc.VectorSubcoreMesh(core_axis_name="c", subcore_axis_name="s"),
        out_type=jax.ShapeDtypeStruct((_NTILES, 16), jnp.float32),
        scratch_types=[
            pltpu.VMEM((2 * _B,), jnp.float32),
            pltpu.VMEM((_ROWS_PER_DMA, _W), jnp.float32),
            pltpu.VMEM((18 * 16,), jnp.float32),
            pltpu.VMEM((18 * 16,), jnp.float32),
            pltpu.VMEM((16,), jnp.float32),
        ],
        compiler_params=pltpu.CompilerParams(needs_layout_passes=False),
    )


_sc_bins_lo = _make_sc(0)
_sc_bins_hi = _make_sc(_HALF)


def kernel(mask, bbox):
    m = mask.reshape(_B, _H, _W)
    cx = bbox[:, 0] * _W
    cy = bbox[:, 1] * _H
    cc = jnp.concatenate([cx, cy])
    p_lo = _prefix_sums(m, 0)
    p_hi = _prefix_sums(m, _HALF)
    pens_lo = _sc_bins_lo(p_lo, cc)
    pens_hi = _sc_bins_hi(p_hi, cc)
    total = jnp.sum(pens_lo) + jnp.sum(pens_hi)
    return _PENALTY_WEIGHT * (total / jnp.float32(_B))


# speedup vs baseline: 1.2605x; 1.1460x over previous
import functools

import jax
import jax.numpy as jnp
import numpy as np
from jax import lax
from jax.experimental import pallas as pl
from jax.experimental.pallas import tpu as pltpu
from jax.experimental.pallas import tpu_sc as plsc

_NUM_BINS = 36
_MIN_ACTIVATION = 0.1
_PENALTY_WEIGHT = 1.0
_H = 384
_W = 384
_B = 64
_NTILES = 32
_HALF = _B // 2
_ROWS_PER_DMA = 64
_NDMA = _H // _ROWS_PER_DMA

_COT = tuple(
    float(np.float32(np.cos((j - 18) * np.pi / 18.0) / np.sin((j - 18) * np.pi / 18.0)))
    for j in range(1, 18)
)


def _take16(v, idx):
    return lax.gather(
        v,
        idx[:, None],
        lax.GatherDimensionNumbers(
            offset_dims=(), collapsed_slice_dims=(0,), start_index_map=(0,)
        ),
        slice_sizes=(1,),
        mode=lax.GatherScatterMode.PROMISE_IN_BOUNDS,
    )


def _cumsum_body(m_ref, p_ref):
    x = m_ref[0] - jnp.float32(_MIN_ACTIVATION)
    r = lax.broadcasted_iota(jnp.int32, (_W, _W), 0)
    c = lax.broadcasted_iota(jnp.int32, (_W, _W), 1)
    u = (r <= c).astype(jnp.bfloat16)
    hi = x.astype(jnp.bfloat16)
    r1 = x - hi.astype(jnp.float32)
    mid = r1.astype(jnp.bfloat16)
    lo = (r1 - mid.astype(jnp.float32)).astype(jnp.bfloat16)
    dims = (((1,), (0,)), ((), ()))
    acc = lax.dot_general(lo, u, dims, preferred_element_type=jnp.float32)
    acc = acc + lax.dot_general(mid, u, dims, preferred_element_type=jnp.float32)
    acc = acc + lax.dot_general(hi, u, dims, preferred_element_type=jnp.float32)
    p_ref[...] = acc


def _prefix_sums(m, base):
    return pl.pallas_call(
        _cumsum_body,
        grid=(_HALF,),
        in_specs=[pl.BlockSpec((1, _H, _W), lambda b: (base + b, 0, 0))],
        out_specs=pl.BlockSpec((_H, _W), lambda b: (b, 0)),
        out_shape=jax.ShapeDtypeStruct((_HALF * _H, _W), jnp.float32),
    )(m)


def _sc_body(base, p_hbm, cc_hbm, out_hbm, cc_v, pbuf, acc_hi, acc_lo, pen_v):
    wid = lax.axis_index("s") * 2 + lax.axis_index("c")
    lane = lax.iota(jnp.int32, 16)
    zeros16 = jnp.zeros((16,), jnp.float32)

    pltpu.sync_copy(cc_hbm, cc_v)

    b = base + wid
    bl = jnp.zeros((16,), jnp.int32) + (b % 16)
    cx = _take16(cc_v[pl.ds((b // 16) * 16, 16)], bl)
    cy = _take16(cc_v[pl.ds(_B + (b // 16) * 16, 16)], bl)

    for j in range(18):
        acc_hi[pl.ds(j * 16, 16)] = zeros16
        acc_lo[pl.ds(j * 16, 16)] = zeros16

    def group_body(g, carry):
        pltpu.sync_copy(
            p_hbm.at[pl.ds(wid * _H + g * _ROWS_PER_DMA, _ROWS_PER_DMA), :], pbuf
        )
        for s in range(_ROWS_PER_DMA // 16):
            row = s * 16 + lane
            yv = (g * _ROWS_PER_DMA + s * 16 + lane).astype(jnp.float32)
            dy = yv - cy
            use_floor = dy > 0.0
            high = dy >= 0.0
            tot = plsc.load_gather(pbuf, [row, jnp.zeros((16,), jnp.int32) + (_W - 1)])
            g_prev = jnp.where(high, tot, 0.0)
            for j in range(1, 19):
                if j < 18:
                    wj = cx + dy * _COT[j - 1]
                    t0 = wj.astype(jnp.int32)
                    tf = t0.astype(jnp.float32)
                    ifl = t0 - (tf > wj).astype(jnp.int32)
                    icl = t0 + (tf < wj).astype(jnp.int32) - 1
                    idx = jnp.clip(jnp.where(use_floor, ifl, icl), -1, _W - 1)
                    gj = plsc.load_gather(pbuf, [row, jnp.maximum(idx, 0)])
                    gj = jnp.where(idx < 0, 0.0, gj)
                else:
                    gj = jnp.where(high, 0.0, tot)
                d = g_prev - gj
                plsc.addupdate(acc_hi.at[pl.ds((j - 1) * 16, 16)], jnp.where(high, d, 0.0))
                plsc.addupdate(acc_lo.at[pl.ds((j - 1) * 16, 16)], jnp.where(high, 0.0, -d))
                g_prev = gj
        return carry

    lax.fori_loop(0, _NDMA, group_body, 0)

    last = jnp.zeros((16,), jnp.int32) + 15
    under_vec = zeros16
    for acc in (acc_hi, acc_lo):
        for j in range(18):
            u = _take16(plsc.cumsum(acc[pl.ds(j * 16, 16)]), last)
            under_vec += jnp.where(u <= 0.0, 1.0, 0.0)
    pen_v[...] = jnp.where(lane == 0, under_vec / jnp.float32(_NUM_BINS), zeros16)
    pltpu.sync_copy(pen_v, out_hbm.at[wid])


def _make_sc(base):
    return pl.kernel(
        functools.partial(_sc_body, base),
        mesh=plsc.VectorSubcoreMesh(core_axis_name="c", subcore_axis_name="s"),
        out_type=jax.ShapeDtypeStruct((_NTILES, 16), jnp.float32),
        scratch_types=[
            pltpu.VMEM((2 * _B,), jnp.float32),
            pltpu.VMEM((_ROWS_PER_DMA, _W), jnp.float32),
            pltpu.VMEM((18 * 16,), jnp.float32),
            pltpu.VMEM((18 * 16,), jnp.float32),
            pltpu.VMEM((16,), jnp.float32),
        ],
        compiler_params=pltpu.CompilerParams(needs_layout_passes=False),
    )


_sc_bins_lo = _make_sc(0)
_sc_bins_hi = _make_sc(_HALF)


def kernel(mask, bbox):
    m = mask.reshape(_B, _H, _W)
    cx = bbox[:, 0] * _W
    cy = bbox[:, 1] * _H
    cc = jnp.concatenate([cx, cy])
    p_lo = _prefix_sums(m, 0)
    p_hi = _prefix_sums(m, _HALF)
    pens_lo = _sc_bins_lo(p_lo, cc)
    pens_hi = _sc_bins_hi(p_hi, cc)
    total = jnp.sum(pens_lo) + jnp.sum(pens_hi)
    return _PENALTY_WEIGHT * (total / jnp.float32(_B))


# speedup vs baseline: 1.2889x; 1.0225x over previous
import functools

import jax
import jax.numpy as jnp
import numpy as np
from jax import lax
from jax.experimental import pallas as pl
from jax.experimental.pallas import tpu as pltpu
from jax.experimental.pallas import tpu_sc as plsc

_NUM_BINS = 36
_MIN_ACTIVATION = 0.1
_PENALTY_WEIGHT = 1.0
_H = 384
_W = 384
_B = 64
_NTILES = 32
_HALF = _B // 2
_ROWS_PER_DMA = 64
_NDMA = _H // _ROWS_PER_DMA

_COT = tuple(
    float(np.float32(np.cos((j - 18) * np.pi / 18.0) / np.sin((j - 18) * np.pi / 18.0)))
    for j in range(1, 18)
)


def _take16(v, idx):
    return lax.gather(
        v,
        idx[:, None],
        lax.GatherDimensionNumbers(
            offset_dims=(), collapsed_slice_dims=(0,), start_index_map=(0,)
        ),
        slice_sizes=(1,),
        mode=lax.GatherScatterMode.PROMISE_IN_BOUNDS,
    )


def _cumsum_body(m_ref, p_ref):
    x = m_ref[0] - jnp.float32(_MIN_ACTIVATION)
    r = lax.broadcasted_iota(jnp.int32, (_W, _W), 0)
    c = lax.broadcasted_iota(jnp.int32, (_W, _W), 1)
    u = (r <= c).astype(jnp.bfloat16)
    hi = x.astype(jnp.bfloat16)
    r1 = x - hi.astype(jnp.float32)
    mid = r1.astype(jnp.bfloat16)
    lo = (r1 - mid.astype(jnp.float32)).astype(jnp.bfloat16)
    dims = (((1,), (0,)), ((), ()))
    acc = lax.dot_general(lo, u, dims, preferred_element_type=jnp.float32)
    acc = acc + lax.dot_general(mid, u, dims, preferred_element_type=jnp.float32)
    acc = acc + lax.dot_general(hi, u, dims, preferred_element_type=jnp.float32)
    p_ref[...] = acc


def _prefix_sums(m, base):
    return pl.pallas_call(
        _cumsum_body,
        grid=(_HALF,),
        in_specs=[pl.BlockSpec((1, _H, _W), lambda b: (base + b, 0, 0))],
        out_specs=pl.BlockSpec((_H, _W), lambda b: (b, 0)),
        out_shape=jax.ShapeDtypeStruct((_HALF * _H, _W), jnp.float32),
    )(m)


def _sc_body(base, p_hbm, cc_hbm, out_hbm, cc_v, pbufa, pbufb, acc_hi, acc_lo, pen_v, sema, semb):
    wid = lax.axis_index("s") * 2 + lax.axis_index("c")
    lane = lax.iota(jnp.int32, 16)
    zeros16 = jnp.zeros((16,), jnp.float32)

    pltpu.sync_copy(cc_hbm, cc_v)

    b = base + wid
    bl = jnp.zeros((16,), jnp.int32) + (b % 16)
    cx = _take16(cc_v[pl.ds((b // 16) * 16, 16)], bl)
    cy = _take16(cc_v[pl.ds(_B + (b // 16) * 16, 16)], bl)

    for j in range(18):
        acc_hi[pl.ds(j * 16, 16)] = zeros16
        acc_lo[pl.ds(j * 16, 16)] = zeros16

    def src(g):
        return p_hbm.at[pl.ds(wid * _H + g * _ROWS_PER_DMA, _ROWS_PER_DMA), :]

    def process(g, buf):
        for s in range(_ROWS_PER_DMA // 16):
            row = s * 16 + lane
            yv = (g * _ROWS_PER_DMA + s * 16 + lane).astype(jnp.float32)
            dy = yv - cy
            use_floor = dy > 0.0
            high = dy >= 0.0
            tot = plsc.load_gather(buf, [row, jnp.zeros((16,), jnp.int32) + (_W - 1)])
            g_prev = jnp.where(high, tot, 0.0)
            for j in range(1, 19):
                if j < 18:
                    wj = cx + dy * _COT[j - 1]
                    t0 = wj.astype(jnp.int32)
                    tf = t0.astype(jnp.float32)
                    ifl = t0 - (tf > wj).astype(jnp.int32)
                    icl = t0 + (tf < wj).astype(jnp.int32) - 1
                    idx = jnp.clip(jnp.where(use_floor, ifl, icl), -1, _W - 1)
                    gj = plsc.load_gather(buf, [row, jnp.maximum(idx, 0)])
                    gj = jnp.where(idx < 0, 0.0, gj)
                else:
                    gj = jnp.where(high, 0.0, tot)
                d = g_prev - gj
                plsc.addupdate(acc_hi.at[pl.ds((j - 1) * 16, 16)], jnp.where(high, d, 0.0))
                plsc.addupdate(acc_lo.at[pl.ds((j - 1) * 16, 16)], jnp.where(high, 0.0, -d))
                g_prev = gj

    pltpu.async_copy(src(0), pbufa, sema)

    def pair_body(i, carry):
        g0 = i * 2
        pltpu.make_async_copy(src(g0), pbufa, sema).wait()
        pltpu.async_copy(src(g0 + 1), pbufb, semb)
        process(g0, pbufa)
        pltpu.make_async_copy(src(g0 + 1), pbufb, semb).wait()
        nxt = jnp.minimum(g0 + 2, _NDMA - 1)
        pltpu.async_copy(src(nxt), pbufa, sema)
        process(g0 + 1, pbufb)
        return carry

    lax.fori_loop(0, _NDMA // 2, pair_body, 0)
    pltpu.make_async_copy(src(_NDMA - 1), pbufa, sema).wait()

    last = jnp.zeros((16,), jnp.int32) + 15
    under_vec = zeros16
    for acc in (acc_hi, acc_lo):
        for j in range(18):
            u = _take16(plsc.cumsum(acc[pl.ds(j * 16, 16)]), last)
            under_vec += jnp.where(u <= 0.0, 1.0, 0.0)
    pen_v[...] = jnp.where(lane == 0, under_vec / jnp.float32(_NUM_BINS), zeros16)
    pltpu.sync_copy(pen_v, out_hbm.at[wid])


def _make_sc(base):
    return pl.kernel(
        functools.partial(_sc_body, base),
        mesh=plsc.VectorSubcoreMesh(core_axis_name="c", subcore_axis_name="s"),
        out_type=jax.ShapeDtypeStruct((_NTILES, 16), jnp.float32),
        scratch_types=[
            pltpu.VMEM((2 * _B,), jnp.float32),
            pltpu.VMEM((_ROWS_PER_DMA, _W), jnp.float32),
            pltpu.VMEM((_ROWS_PER_DMA, _W), jnp.float32),
            pltpu.VMEM((18 * 16,), jnp.float32),
            pltpu.VMEM((18 * 16,), jnp.float32),
            pltpu.VMEM((16,), jnp.float32),
            pltpu.SemaphoreType.DMA,
            pltpu.SemaphoreType.DMA,
        ],
        compiler_params=pltpu.CompilerParams(needs_layout_passes=False),
    )


_sc_bins_lo = _make_sc(0)
_sc_bins_hi = _make_sc(_HALF)


def kernel(mask, bbox):
    m = mask.reshape(_B, _H, _W)
    cx = bbox[:, 0] * _W
    cy = bbox[:, 1] * _H
    cc = jnp.concatenate([cx, cy])
    p_lo = _prefix_sums(m, 0)
    p_hi = _prefix_sums(m, _HALF)
    pens_lo = _sc_bins_lo(p_lo, cc)
    pens_hi = _sc_bins_hi(p_hi, cc)
    total = jnp.sum(pens_lo) + jnp.sum(pens_hi)
    return _PENALTY_WEIGHT * (total / jnp.float32(_B))
